# baseline (speedup 1.0000x reference)
import functools

import jax
import jax.numpy as jnp
from jax import lax
from jax.experimental import pallas as pl
from jax.experimental.pallas import tpu as pltpu
from jax.experimental.pallas import tpu_sc as plsc

T, B, H, N = 4, 4, 4, 512
S = 32
NB = N // S
NPAIR = NB * (NB + 1) // 2
NUNITS = NPAIR * B
NW = 32
UPW = NUNITS // NW
OFF = [0, 16, 31, 45, 58, 70, 81, 91, 100, 108, 115, 121, 126, 130, 133, 135]

_LANES = 16
_NC = S // _LANES


def _sc_body(adj_set_hbm, adj_static_hbm, out_static_hbm, out_set_hbm,
             a_buf, g_buf, set_buf):
    wid = lax.axis_index("s") * 2 + lax.axis_index("c")
    lanes = lax.broadcasted_iota(jnp.int32, (_LANES,), 0)

    def unit_body(k, _):
        u = wid + k * NW
        b = u & 3
        p = u >> 2
        i = jnp.int32(0)
        off_sel = jnp.int32(0)
        for ii in range(1, NB):
            ge = p >= OFF[ii]
            i = jnp.where(ge, jnp.int32(ii), i)
            off_sel = jnp.where(ge, jnp.int32(OFF[ii]), off_sel)
        j = i + (p - off_sel)
        is_diag = i == j
        r0 = i * S
        c0 = j * S

        pltpu.sync_copy(adj_static_hbm.at[b, :, pl.ds(r0, S), pl.ds(c0, S)],
                        g_buf.at[0])
        @pl.when(jnp.logical_not(is_diag))
        def _():
            pltpu.sync_copy(adj_static_hbm.at[b, :, pl.ds(c0, S), pl.ds(r0, S)],
                            g_buf.at[1])

        for t in range(T):
            pltpu.sync_copy(adj_set_hbm.at[t, b, :, pl.ds(r0, S), pl.ds(c0, S)],
                            a_buf.at[0])
            @pl.when(jnp.logical_not(is_diag))
            def _():
                pltpu.sync_copy(
                    adj_set_hbm.at[t, b, :, pl.ds(c0, S), pl.ds(r0, S)],
                    a_buf.at[1])

            def p1(n, _):
                for blk in range(2):
                    src_blk = jnp.where(is_diag, jnp.int32(0),
                                        jnp.int32(1 - blk))
                    def do_blk(blk=blk, src_blk=src_blk):
                        for h in range(H):
                            for c in range(_NC):
                                m = c * _LANES + lanes
                                a = a_buf[blk, h, n, pl.ds(c * _LANES, _LANES)]
                                g = g_buf[blk, h, n, pl.ds(c * _LANES, _LANES)]
                                gt = plsc.load_gather(
                                    g_buf,
                                    [jnp.full((_LANES,), src_blk, jnp.int32),
                                     jnp.full((_LANES,), h, jnp.int32),
                                     m,
                                     jnp.full((_LANES,), n, jnp.int32)])
                                eye = jnp.where(
                                    jnp.logical_and(is_diag, m == n),
                                    jnp.float32(1.0), jnp.float32(0.0))
                                ssum = g + gt + eye
                                setv = jnp.where(ssum > 0, a, jnp.float32(0.0))
                                set_buf[blk, h, n,
                                        pl.ds(c * _LANES, _LANES)] = setv
                    if blk == 0:
                        do_blk()
                    else:
                        pl.when(jnp.logical_not(is_diag))(do_blk)
                return 0
            lax.fori_loop(0, S, p1, 0)

            pltpu.sync_copy(set_buf.at[0],
                            out_set_hbm.at[t, b, :, pl.ds(r0, S), pl.ds(c0, S)])
            @pl.when(jnp.logical_not(is_diag))
            def _():
                pltpu.sync_copy(
                    set_buf.at[1],
                    out_set_hbm.at[t, b, :, pl.ds(c0, S), pl.ds(r0, S)])

            def p2(n, _):
                for blk in range(2):
                    def do_blk(blk=blk):
                        for c in range(_NC):
                            sl = pl.ds(c * _LANES, _LANES)
                            av = [a_buf[blk, h, n, sl] for h in range(H)]
                            maxa = av[0]
                            for h in range(1, H):
                                maxa = jnp.maximum(maxa, av[h])
                            for h in range(H):
                                keep = av[h] <= maxa
                                g = g_buf[blk, h, n, sl]
                                g_buf[blk, h, n, sl] = jnp.where(
                                    keep, g, jnp.float32(0.0))
                    if blk == 0:
                        do_blk()
                    else:
                        pl.when(jnp.logical_not(is_diag))(do_blk)
                return 0
            lax.fori_loop(0, S, p2, 0)

            pltpu.sync_copy(g_buf.at[0],
                            out_static_hbm.at[t, b, :,
                                              pl.ds(r0, S), pl.ds(c0, S)])
            @pl.when(jnp.logical_not(is_diag))
            def _():
                pltpu.sync_copy(
                    g_buf.at[1],
                    out_static_hbm.at[t, b, :, pl.ds(c0, S), pl.ds(r0, S)])
        return 0

    lax.fori_loop(0, UPW, unit_body, 0)


@jax.jit
def kernel(adj_set, adj_static):
    out = jax.ShapeDtypeStruct((T, B, H, N, N), jnp.float32)
    mesh = plsc.VectorSubcoreMesh(core_axis_name="c", subcore_axis_name="s")
    f = functools.partial(
        pl.kernel, mesh=mesh,
        compiler_params=pltpu.CompilerParams(use_tc_tiling_on_sc=False,
                                             needs_layout_passes=False),
        out_type=[out, out],
        scratch_types=[
            pltpu.VMEM((2, H, S, S), jnp.float32),
            pltpu.VMEM((2, H, S, S), jnp.float32),
            pltpu.VMEM((2, H, S, S), jnp.float32),
        ],
    )(_sc_body)
    return tuple(f(adj_set, adj_static))


# speedup vs baseline: 1.1481x; 1.1481x over previous
import functools

import jax
import jax.numpy as jnp
from jax import lax
from jax.experimental import pallas as pl
from jax.experimental.pallas import tpu as pltpu
from jax.experimental.pallas import tpu_sc as plsc

T, B, H, N = 4, 4, 4, 512
S = 32
NB = N // S
NPAIR = NB * (NB + 1) // 2
NUNITS = NPAIR * B
NW = 32
UPW = NUNITS // NW
OFF = [0, 16, 31, 45, 58, 70, 81, 91, 100, 108, 115, 121, 126, 130, 133, 135]

_LANES = 16
_NC = S // _LANES


def _sc_body(adj_set_hbm, adj_static_hbm, out_static_hbm, out_set_hbm,
             a_buf, g_buf, set_buf):
    wid = lax.axis_index("s") * 2 + lax.axis_index("c")
    lanes = lax.broadcasted_iota(jnp.int32, (_LANES,), 0)

    def unit_body(k, _):
        u = wid + k * NW
        b = u & 3
        p = u >> 2
        i = jnp.int32(0)
        off_sel = jnp.int32(0)
        for ii in range(1, NB):
            ge = p >= OFF[ii]
            i = jnp.where(ge, jnp.int32(ii), i)
            off_sel = jnp.where(ge, jnp.int32(OFF[ii]), off_sel)
        j = i + (p - off_sel)
        is_diag = i == j
        r0 = i * S
        c0 = j * S

        pltpu.sync_copy(adj_static_hbm.at[b, :, pl.ds(r0, S), pl.ds(c0, S)],
                        g_buf.at[0, :, :, pl.ds(0, S)])
        @pl.when(jnp.logical_not(is_diag))
        def _():
            pltpu.sync_copy(adj_static_hbm.at[b, :, pl.ds(c0, S), pl.ds(r0, S)],
                            g_buf.at[1, :, :, pl.ds(0, S)])

        for t in range(T):
            pltpu.sync_copy(adj_set_hbm.at[t, b, :, pl.ds(r0, S), pl.ds(c0, S)],
                            a_buf.at[0])
            @pl.when(jnp.logical_not(is_diag))
            def _():
                pltpu.sync_copy(
                    adj_set_hbm.at[t, b, :, pl.ds(c0, S), pl.ds(r0, S)],
                    a_buf.at[1])

            def p1(n, _):
                for blk in range(2):
                    src_blk = jnp.where(is_diag, jnp.int32(0),
                                        jnp.int32(1 - blk))
                    def do_blk(blk=blk, src_blk=src_blk):
                        for h in range(H):
                            for c in range(_NC):
                                m = c * _LANES + lanes
                                a = a_buf[blk, h, n, pl.ds(c * _LANES, _LANES)]
                                g = g_buf[blk, h, n, pl.ds(c * _LANES, _LANES)]
                                gt = plsc.load_gather(
                                    g_buf,
                                    [jnp.full((_LANES,), src_blk, jnp.int32),
                                     jnp.full((_LANES,), h, jnp.int32),
                                     m,
                                     jnp.full((_LANES,), n, jnp.int32)])
                                eye = jnp.where(
                                    jnp.logical_and(is_diag, m == n),
                                    jnp.float32(1.0), jnp.float32(0.0))
                                ssum = g + gt + eye
                                setv = jnp.where(ssum > 0, a, jnp.float32(0.0))
                                set_buf[blk, h, n,
                                        pl.ds(c * _LANES, _LANES)] = setv
                    if blk == 0:
                        do_blk()
                    else:
                        pl.when(jnp.logical_not(is_diag))(do_blk)
                return 0
            lax.fori_loop(0, S, p1, 0)

            pltpu.sync_copy(set_buf.at[0],
                            out_set_hbm.at[t, b, :, pl.ds(r0, S), pl.ds(c0, S)])
            @pl.when(jnp.logical_not(is_diag))
            def _():
                pltpu.sync_copy(
                    set_buf.at[1],
                    out_set_hbm.at[t, b, :, pl.ds(c0, S), pl.ds(r0, S)])

            def p2(n, _):
                for blk in range(2):
                    def do_blk(blk=blk):
                        for c in range(_NC):
                            sl = pl.ds(c * _LANES, _LANES)
                            av = [a_buf[blk, h, n, sl] for h in range(H)]
                            maxa = av[0]
                            for h in range(1, H):
                                maxa = jnp.maximum(maxa, av[h])
                            for h in range(H):
                                keep = av[h] <= maxa
                                g = g_buf[blk, h, n, sl]
                                g_buf[blk, h, n, sl] = jnp.where(
                                    keep, g, jnp.float32(0.0))
                    if blk == 0:
                        do_blk()
                    else:
                        pl.when(jnp.logical_not(is_diag))(do_blk)
                return 0
            lax.fori_loop(0, S, p2, 0)

            pltpu.sync_copy(g_buf.at[0, :, :, pl.ds(0, S)],
                            out_static_hbm.at[t, b, :,
                                              pl.ds(r0, S), pl.ds(c0, S)])
            @pl.when(jnp.logical_not(is_diag))
            def _():
                pltpu.sync_copy(
                    g_buf.at[1, :, :, pl.ds(0, S)],
                    out_static_hbm.at[t, b, :, pl.ds(c0, S), pl.ds(r0, S)])
        return 0

    lax.fori_loop(0, UPW, unit_body, 0)


@jax.jit
def kernel(adj_set, adj_static):
    out = jax.ShapeDtypeStruct((T, B, H, N, N), jnp.float32)
    mesh = plsc.VectorSubcoreMesh(core_axis_name="c", subcore_axis_name="s")
    f = functools.partial(
        pl.kernel, mesh=mesh,
        compiler_params=pltpu.CompilerParams(use_tc_tiling_on_sc=False,
                                             needs_layout_passes=False),
        out_type=[out, out],
        scratch_types=[
            pltpu.VMEM((2, H, S, S), jnp.float32),
            pltpu.VMEM((2, H, S, S + 1), jnp.float32),
            pltpu.VMEM((2, H, S, S), jnp.float32),
        ],
    )(_sc_body)
    return tuple(f(adj_set, adj_static))


# speedup vs baseline: 1.2540x; 1.0922x over previous
import functools

import jax
import jax.numpy as jnp
from jax import lax
from jax.experimental import pallas as pl
from jax.experimental.pallas import tpu as pltpu
from jax.experimental.pallas import tpu_sc as plsc

T, B, H, N = 4, 4, 4, 512
S = 64
NB = N // S
NPAIR = NB * (NB + 1) // 2
NUNITS = NPAIR * B
NW = 32
UPW = -(-NUNITS // NW)
OFF = [0, 8, 15, 21, 26, 30, 33, 35]

_LANES = 16
_NC = S // _LANES


def _sc_body(adj_set_hbm, adj_static_hbm, out_static_hbm, out_set_hbm,
             a_buf, g_buf, set_buf):
    wid = lax.axis_index("s") * 2 + lax.axis_index("c")
    lanes = lax.broadcasted_iota(jnp.int32, (_LANES,), 0)

    def unit_body(k, _):
        u = wid + k * NW

        @pl.when(u < NUNITS)
        def _unit():
            _unit_inner(u)
        return 0

    def _unit_inner(u):
        b = u & 3
        p = u >> 2
        i = jnp.int32(0)
        off_sel = jnp.int32(0)
        for ii in range(1, NB):
            ge = p >= OFF[ii]
            i = jnp.where(ge, jnp.int32(ii), i)
            off_sel = jnp.where(ge, jnp.int32(OFF[ii]), off_sel)
        j = i + (p - off_sel)
        is_diag = i == j
        r0 = i * S
        c0 = j * S

        pltpu.sync_copy(adj_static_hbm.at[b, :, pl.ds(r0, S), pl.ds(c0, S)],
                        g_buf.at[0, :, :, pl.ds(0, S)])
        @pl.when(jnp.logical_not(is_diag))
        def _():
            pltpu.sync_copy(adj_static_hbm.at[b, :, pl.ds(c0, S), pl.ds(r0, S)],
                            g_buf.at[1, :, :, pl.ds(0, S)])

        for t in range(T):
            pltpu.sync_copy(adj_set_hbm.at[t, b, :, pl.ds(r0, S), pl.ds(c0, S)],
                            a_buf.at[0])
            @pl.when(jnp.logical_not(is_diag))
            def _():
                pltpu.sync_copy(
                    adj_set_hbm.at[t, b, :, pl.ds(c0, S), pl.ds(r0, S)],
                    a_buf.at[1])

            def p1(n, _):
                for blk in range(2):
                    src_blk = jnp.where(is_diag, jnp.int32(0),
                                        jnp.int32(1 - blk))
                    def do_blk(blk=blk, src_blk=src_blk):
                        for h in range(H):
                            for c in range(_NC):
                                m = c * _LANES + lanes
                                a = a_buf[blk, h, n, pl.ds(c * _LANES, _LANES)]
                                g = g_buf[blk, h, n, pl.ds(c * _LANES, _LANES)]
                                gt = plsc.load_gather(
                                    g_buf,
                                    [jnp.full((_LANES,), src_blk, jnp.int32),
                                     jnp.full((_LANES,), h, jnp.int32),
                                     m,
                                     jnp.full((_LANES,), n, jnp.int32)])
                                eye = jnp.where(
                                    jnp.logical_and(is_diag, m == n),
                                    jnp.float32(1.0), jnp.float32(0.0))
                                ssum = g + gt + eye
                                setv = jnp.where(ssum > 0, a, jnp.float32(0.0))
                                set_buf[blk, h, n,
                                        pl.ds(c * _LANES, _LANES)] = setv
                    if blk == 0:
                        do_blk()
                    else:
                        pl.when(jnp.logical_not(is_diag))(do_blk)
                return 0
            lax.fori_loop(0, S, p1, 0)

            pltpu.sync_copy(set_buf.at[0],
                            out_set_hbm.at[t, b, :, pl.ds(r0, S), pl.ds(c0, S)])
            @pl.when(jnp.logical_not(is_diag))
            def _():
                pltpu.sync_copy(
                    set_buf.at[1],
                    out_set_hbm.at[t, b, :, pl.ds(c0, S), pl.ds(r0, S)])

            def p2(n, _):
                for blk in range(2):
                    def do_blk(blk=blk):
                        for c in range(_NC):
                            sl = pl.ds(c * _LANES, _LANES)
                            av = [a_buf[blk, h, n, sl] for h in range(H)]
                            maxa = av[0]
                            for h in range(1, H):
                                maxa = jnp.maximum(maxa, av[h])
                            for h in range(H):
                                keep = av[h] <= maxa
                                g = g_buf[blk, h, n, sl]
                                g_buf[blk, h, n, sl] = jnp.where(
                                    keep, g, jnp.float32(0.0))
                    if blk == 0:
                        do_blk()
                    else:
                        pl.when(jnp.logical_not(is_diag))(do_blk)
                return 0
            lax.fori_loop(0, S, p2, 0)

            pltpu.sync_copy(g_buf.at[0, :, :, pl.ds(0, S)],
                            out_static_hbm.at[t, b, :,
                                              pl.ds(r0, S), pl.ds(c0, S)])
            @pl.when(jnp.logical_not(is_diag))
            def _():
                pltpu.sync_copy(
                    g_buf.at[1, :, :, pl.ds(0, S)],
                    out_static_hbm.at[t, b, :, pl.ds(c0, S), pl.ds(r0, S)])

    lax.fori_loop(0, UPW, unit_body, 0)


@jax.jit
def kernel(adj_set, adj_static):
    out = jax.ShapeDtypeStruct((T, B, H, N, N), jnp.float32)
    mesh = plsc.VectorSubcoreMesh(core_axis_name="c", subcore_axis_name="s")
    f = functools.partial(
        pl.kernel, mesh=mesh,
        compiler_params=pltpu.CompilerParams(use_tc_tiling_on_sc=False,
                                             needs_layout_passes=False),
        out_type=[out, out],
        scratch_types=[
            pltpu.VMEM((2, H, S, S), jnp.float32),
            pltpu.VMEM((2, H, S, S + 1), jnp.float32),
            pltpu.VMEM((2, H, S, S), jnp.float32),
        ],
    )(_sc_body)
    return tuple(f(adj_set, adj_static))


# speedup vs baseline: 12.4112x; 9.8974x over previous
import functools

import jax
import jax.numpy as jnp
from jax.experimental import pallas as pl
from jax.experimental.pallas import tpu as pltpu

CSPLIT = 2


def _body(a_ref, s0_ref, out_static_ref, out_set_ref, cum_ref):
    t = pl.program_id(1)
    c = pl.program_id(2)
    n = s0_ref.shape[-1]
    w = n // CSPLIT

    a = a_ref[0, 0]
    s0c = s0_ref[0, :, :, pl.ds(c * w, w)]
    s0r = s0_ref[0, :, pl.ds(c * w, w), :]
    prev = (t + 1) % 2
    cur = t % 2
    ones = jnp.float32(1.0)

    cpc = jnp.where(t == 0, ones, cum_ref[prev, :, :, pl.ds(c * w, w)])
    cpr = jnp.where(t == 0, ones, cum_ref[prev, :, pl.ds(c * w, w), :])
    g = s0c * cpc
    gr = s0r * cpr
    gt = jnp.swapaxes(gr, 1, 2)

    row = jax.lax.broadcasted_iota(jnp.int32, (n, w), 0)
    col = jax.lax.broadcasted_iota(jnp.int32, (n, w), 1) + c * w
    eye = jnp.where(row == col, ones, jnp.float32(0.0))
    mask_d = (g + gt + eye[None]) > 0
    out_set_ref[0, 0] = jnp.where(mask_d, a, jnp.float32(0.0))

    maxa = jnp.max(a, axis=0, keepdims=True)
    keep = a <= maxa
    cnew = jnp.where(keep, cpc, jnp.float32(0.0))
    cum_ref[cur, :, :, pl.ds(c * w, w)] = cnew
    out_static_ref[0, 0] = s0c * cnew


@functools.partial(jax.jit, static_argnames=())
def kernel(adj_set, adj_static):
    T, B, H, N, _ = adj_set.shape
    W = N // CSPLIT
    out_shape = jax.ShapeDtypeStruct((T, B, H, N, N), adj_set.dtype)
    grid = (B, T, CSPLIT)
    out_static, out_set = pl.pallas_call(
        _body,
        grid=grid,
        in_specs=[
            pl.BlockSpec((1, 1, H, N, W), lambda b, t, c: (t, b, 0, 0, c)),
            pl.BlockSpec((1, H, N, N), lambda b, t, c: (b, 0, 0, 0)),
        ],
        out_specs=[
            pl.BlockSpec((1, 1, H, N, W), lambda b, t, c: (t, b, 0, 0, c)),
            pl.BlockSpec((1, 1, H, N, W), lambda b, t, c: (t, b, 0, 0, c)),
        ],
        out_shape=[out_shape, out_shape],
        scratch_shapes=[pltpu.VMEM((2, H, N, N), jnp.float32)],
        compiler_params=pltpu.CompilerParams(
            dimension_semantics=("arbitrary", "arbitrary", "arbitrary"),
        ),
    )(adj_set, adj_static)
    return out_static, out_set


# speedup vs baseline: 13.6263x; 1.0979x over previous
import functools

import jax
import jax.numpy as jnp
from jax.experimental import pallas as pl
from jax.experimental.pallas import tpu as pltpu


def _body(a_ref, s0_ref, out_static_ref, out_set_ref, cum_ref):
    t = pl.program_id(1)

    @pl.when(t == 0)
    def _init():
        cum_ref[...] = jnp.ones_like(cum_ref)

    a = a_ref[0, 0]
    s0 = s0_ref[0]
    cum = cum_ref[...]

    g = s0 * cum
    gt = jnp.swapaxes(g, 1, 2)
    n = g.shape[-1]
    row = jax.lax.broadcasted_iota(jnp.int32, (n, n), 0)
    col = jax.lax.broadcasted_iota(jnp.int32, (n, n), 1)
    eye = jnp.where(row == col, jnp.float32(1.0), jnp.float32(0.0))
    adj_sum = g + gt + eye[None]
    mask_d = jnp.where(adj_sum > 0, jnp.float32(1.0), jnp.float32(0.0))
    out_set_ref[0, 0] = a * mask_d

    maxa = jnp.max(a, axis=0, keepdims=True)
    keep = jnp.where(a <= maxa, jnp.float32(1.0), jnp.float32(0.0))
    cum = cum * keep
    cum_ref[...] = cum
    out_static_ref[0, 0] = s0 * cum


@functools.partial(jax.jit, static_argnames=())
def kernel(adj_set, adj_static):
    T, B, H, N, _ = adj_set.shape
    out_shape = jax.ShapeDtypeStruct((T, B, H, N, N), adj_set.dtype)
    grid = (B, T)
    out_static, out_set = pl.pallas_call(
        _body,
        grid=grid,
        in_specs=[
            pl.BlockSpec((1, 1, H, N, N), lambda b, t: (t, b, 0, 0, 0)),
            pl.BlockSpec((1, H, N, N), lambda b, t: (b, 0, 0, 0)),
        ],
        out_specs=[
            pl.BlockSpec((1, 1, H, N, N), lambda b, t: (t, b, 0, 0, 0)),
            pl.BlockSpec((1, 1, H, N, N), lambda b, t: (t, b, 0, 0, 0)),
        ],
        out_shape=[out_shape, out_shape],
        scratch_shapes=[pltpu.VMEM((H, N, N), jnp.float32)],
        compiler_params=pltpu.CompilerParams(
            dimension_semantics=("arbitrary", "arbitrary"),
        ),
    )(adj_set, adj_static)
    return out_static, out_set
